# Initial kernel scaffold; baseline (speedup 1.0000x reference)
import jax
import jax.numpy as jnp
from jax.experimental import pallas as pl


def kernel(x, pe):
    raise NotImplementedError("write your pallas kernel here")


# speedup vs baseline: 4.9499x; 4.9499x over previous
import functools

import jax
import jax.numpy as jnp
from jax import lax
from jax.experimental import pallas as pl
from jax.experimental.pallas import tpu as pltpu
from jax.experimental.pallas import tpu_sc as plsc

DIM = 64
CHUNK = 128
NBUF = 4


def _worker_count():
    try:
        info = plsc.get_sparse_core_info()
        return info.num_cores, info.num_subcores
    except Exception:
        return 2, 16


@functools.lru_cache(maxsize=None)
def _build(n_chunks_total):
    num_cores, num_subcores = _worker_count()
    nw = num_cores * num_subcores
    chunks_per_w = n_chunks_total // nw
    groups = chunks_per_w // NBUF
    assert chunks_per_w * nw == n_chunks_total
    assert groups * NBUF == chunks_per_w

    mesh = plsc.VectorSubcoreMesh(core_axis_name="c", subcore_axis_name="s")
    n_rows = n_chunks_total * CHUNK

    @functools.partial(
        pl.kernel,
        out_type=jax.ShapeDtypeStruct((n_rows, DIM), jnp.float32),
        mesh=mesh,
        scratch_types=(
            [pltpu.VMEM((chunks_per_w, CHUNK), jnp.int32)]
            + [pltpu.VMEM((CHUNK, DIM), jnp.float32) for _ in range(NBUF)]
            + [pltpu.SemaphoreType.DMA for _ in range(2 * NBUF)]
        ),
        compiler_params=pltpu.CompilerParams(use_tc_tiling_on_sc=False),
    )
    def gather_kernel(x_hbm, pe_hbm, out_hbm, idx_v, *rest):
        rows = rest[:NBUF]
        gsem = rest[NBUF : 2 * NBUF]
        ssem = rest[2 * NBUF :]

        wid = lax.axis_index("s") * num_cores + lax.axis_index("c")
        cbase = wid * chunks_per_w

        pltpu.sync_copy(x_hbm.at[pl.ds(cbase, chunks_per_w)], idx_v)

        def gather(c, b):
            return pltpu.make_async_copy(
                pe_hbm.at[idx_v.at[c]], rows[b], gsem[b]
            )

        def store(c, b):
            return pltpu.make_async_copy(
                rows[b], out_hbm.at[pl.ds((cbase + c) * CHUNK, CHUNK)], ssem[b]
            )

        for b in range(NBUF):
            gather(b, b).start()

        def body(g, carry):
            c0 = g * NBUF
            for b in range(NBUF):
                gather(c0 + b, b).wait()
                store(c0 + b, b).start()
            for b in range(NBUF):
                c = c0 + b
                store(c, b).wait()

                @pl.when(c + NBUF < chunks_per_w)
                def _():
                    gather(c + NBUF, b).start()

            return carry

        lax.fori_loop(0, groups, body, 0)

    return gather_kernel


def kernel(x, pe):
    b, l = x.shape
    n_total = b * l
    x2 = x.reshape(n_total // CHUNK, CHUNK).astype(jnp.int32)
    out = _build(n_total // CHUNK)(x2, pe)
    return out.reshape(b, l, DIM)


# speedup vs baseline: 4.9552x; 1.0011x over previous
import functools

import jax
import jax.numpy as jnp
from jax import lax
from jax.experimental import pallas as pl
from jax.experimental.pallas import tpu as pltpu
from jax.experimental.pallas import tpu_sc as plsc

DIM = 64
CHUNK = 128
NBUF = 8


def _worker_count():
    try:
        info = plsc.get_sparse_core_info()
        return info.num_cores, info.num_subcores
    except Exception:
        return 2, 16


@functools.lru_cache(maxsize=None)
def _build(n_chunks_total):
    num_cores, num_subcores = _worker_count()
    nw = num_cores * num_subcores
    chunks_per_w = n_chunks_total // nw
    groups = chunks_per_w // NBUF
    assert chunks_per_w * nw == n_chunks_total
    assert groups * NBUF == chunks_per_w

    mesh = plsc.VectorSubcoreMesh(core_axis_name="c", subcore_axis_name="s")
    n_rows = n_chunks_total * CHUNK

    @functools.partial(
        pl.kernel,
        out_type=jax.ShapeDtypeStruct((n_rows, DIM), jnp.float32),
        mesh=mesh,
        scratch_types=(
            [pltpu.VMEM((chunks_per_w, CHUNK), jnp.int32)]
            + [pltpu.VMEM((CHUNK, DIM), jnp.float32) for _ in range(NBUF)]
            + [pltpu.SemaphoreType.DMA for _ in range(2 * NBUF)]
        ),
        compiler_params=pltpu.CompilerParams(use_tc_tiling_on_sc=False),
    )
    def gather_kernel(x_hbm, pe_hbm, out_hbm, idx_v, *rest):
        rows = rest[:NBUF]
        gsem = rest[NBUF : 2 * NBUF]
        ssem = rest[2 * NBUF :]

        wid = lax.axis_index("s") * num_cores + lax.axis_index("c")
        cbase = wid * chunks_per_w

        pltpu.sync_copy(x_hbm.at[pl.ds(cbase, chunks_per_w)], idx_v)

        def gather(c, b):
            return pltpu.make_async_copy(
                pe_hbm.at[idx_v.at[c]], rows[b], gsem[b]
            )

        def store(c, b):
            return pltpu.make_async_copy(
                rows[b], out_hbm.at[pl.ds((cbase + c) * CHUNK, CHUNK)], ssem[b]
            )

        for b in range(NBUF):
            gather(b, b).start()

        def body(g, carry):
            c0 = g * NBUF
            for b in range(NBUF):
                gather(c0 + b, b).wait()
                store(c0 + b, b).start()
            for b in range(NBUF):
                c = c0 + b
                store(c, b).wait()

                @pl.when(c + NBUF < chunks_per_w)
                def _():
                    gather(c + NBUF, b).start()

            return carry

        lax.fori_loop(0, groups, body, 0)

    return gather_kernel


def kernel(x, pe):
    b, l = x.shape
    n_total = b * l
    x2 = x.reshape(n_total // CHUNK, CHUNK).astype(jnp.int32)
    out = _build(n_total // CHUNK)(x2, pe)
    return out.reshape(b, l, DIM)
